# Initial kernel scaffold; baseline (speedup 1.0000x reference)
import jax
import jax.numpy as jnp
from jax.experimental import pallas as pl


def kernel(dR, Z, embeddings):
    raise NotImplementedError("write your pallas kernel here")


# speedup vs baseline: 1.4560x; 1.4560x over previous
import jax
import jax.numpy as jnp
from jax import lax
from jax.experimental import pallas as pl
from jax.experimental.pallas import tpu as pltpu
from jax.experimental.pallas import tpu_sc as plsc

N_ATOMS = 100000
D = 128
NW = 32
PER_W = 3128
B_PAD = NW * PER_W
C = 184
NCH = 17
TAIL = 88


def _gather_body(z_hbm, emb_hbm, out_hbm, idx_v, buf_v, sem):
    wid = lax.axis_index("s") * 2 + lax.axis_index("c")
    base = wid * PER_W
    pltpu.sync_copy(z_hbm.at[pl.ds(base, PER_W)], idx_v)
    for k in range(NCH):
        row0 = base + k * C
        pltpu.async_copy(emb_hbm.at[idx_v.at[pl.ds(k * C, C)]], buf_v, sem).wait()
        if k < NCH - 1:
            pltpu.sync_copy(buf_v, out_hbm.at[pl.ds(row0, C)])
        else:
            @pl.when(wid < NW - 1)
            def _full():
                pltpu.sync_copy(buf_v, out_hbm.at[pl.ds(row0, C)])

            @pl.when(wid == NW - 1)
            def _tail():
                pltpu.sync_copy(buf_v.at[pl.ds(0, TAIL)],
                                out_hbm.at[pl.ds(row0, TAIL)])


def kernel(dR, Z, embeddings):
    del dR
    z_pad = jnp.concatenate(
        [Z, jnp.zeros((B_PAD - N_ATOMS,), jnp.int32)])
    mesh = plsc.VectorSubcoreMesh(core_axis_name="c", subcore_axis_name="s")
    f = pl.kernel(
        _gather_body,
        out_type=jax.ShapeDtypeStruct((N_ATOMS, D), jnp.float32),
        mesh=mesh,
        scratch_types=[
            pltpu.VMEM((PER_W,), jnp.int32),
            pltpu.VMEM((C, D), jnp.float32),
            pltpu.SemaphoreType.DMA,
        ],
    )
    return f(z_pad, embeddings)


# speedup vs baseline: 1.5164x; 1.0415x over previous
import jax
import jax.numpy as jnp
from jax import lax
from jax.experimental import pallas as pl
from jax.experimental.pallas import tpu as pltpu
from jax.experimental.pallas import tpu_sc as plsc

N_ATOMS = 100000
D = 128
NW = 32
PER_W = 3128
B_PAD = NW * PER_W
CMAX = 448
SIZES = (448, 448, 448, 448, 448, 448, 440)
OFFS = (0, 448, 896, 1344, 1792, 2240, 2688)
NCH = len(SIZES)
LAST_TAIL = 344


def _gather_body(z_hbm, emb_hbm, out_hbm,
                 idx_v, buf0, buf1, gs0, gs1, ws0, ws1):
    bufs, gsems, wsems = (buf0, buf1), (gs0, gs1), (ws0, ws1)
    wid = lax.axis_index("s") * 2 + lax.axis_index("c")
    base = wid * PER_W
    pltpu.sync_copy(z_hbm.at[pl.ds(base, PER_W)], idx_v)

    def start_gather(k):
        b, n = k % 2, SIZES[k]
        dst = bufs[b] if n == CMAX else bufs[b].at[pl.ds(0, n)]
        return pltpu.async_copy(
            emb_hbm.at[idx_v.at[pl.ds(OFFS[k], n)]], dst, gsems[b])

    gh = [None, None]
    wh = [None, None]
    gh[0] = start_gather(0)
    for k in range(NCH):
        b, n = k % 2, SIZES[k]
        gh[b].wait()
        if k + 1 < NCH:
            nb = (k + 1) % 2
            if wh[nb] is not None:
                wh[nb].wait()
            gh[nb] = start_gather(k + 1)
        if k < NCH - 1:
            src = bufs[b] if n == CMAX else bufs[b].at[pl.ds(0, n)]
            wh[b] = pltpu.async_copy(
                src, out_hbm.at[pl.ds(base + OFFS[k], n)], wsems[b])
        else:
            @pl.when(wid < NW - 1)
            def _full():
                pltpu.sync_copy(bufs[b].at[pl.ds(0, n)],
                                out_hbm.at[pl.ds(base + OFFS[k], n)])

            @pl.when(wid == NW - 1)
            def _tail():
                pltpu.sync_copy(bufs[b].at[pl.ds(0, LAST_TAIL)],
                                out_hbm.at[pl.ds(base + OFFS[k], LAST_TAIL)])
    wh[(NCH - 2) % 2].wait()


def kernel(dR, Z, embeddings):
    del dR
    z_pad = jnp.concatenate(
        [Z, jnp.zeros((B_PAD - N_ATOMS,), jnp.int32)])
    mesh = plsc.VectorSubcoreMesh(core_axis_name="c", subcore_axis_name="s")
    f = pl.kernel(
        _gather_body,
        out_type=jax.ShapeDtypeStruct((N_ATOMS, D), jnp.float32),
        mesh=mesh,
        scratch_types=[
            pltpu.VMEM((PER_W,), jnp.int32),
            pltpu.VMEM((CMAX, D), jnp.float32),
            pltpu.VMEM((CMAX, D), jnp.float32),
            pltpu.SemaphoreType.DMA,
            pltpu.SemaphoreType.DMA,
            pltpu.SemaphoreType.DMA,
            pltpu.SemaphoreType.DMA,
        ],
    )
    return f(z_pad, embeddings)
